# baseline (speedup 1.0000x reference)
import functools

import jax
import jax.numpy as jnp
from jax.experimental import pallas as pl


def _prep_kernel(a_mem_ref, n_mem_ref, w_mu_ref, w_var_ref,
                 g_a_ref, g_n_ref, h_n_ref):
    dn = (((1,), (1,)), ((), ()))
    g_a_ref[...] = jax.lax.dot_general(
        w_mu_ref[...], a_mem_ref[...], dn, preferred_element_type=jnp.float32)
    g_n_ref[...] = jax.lax.dot_general(
        w_mu_ref[...], n_mem_ref[...], dn, preferred_element_type=jnp.float32)
    h_n_ref[...] = jax.lax.dot_general(
        w_var_ref[...], n_mem_ref[...], dn, preferred_element_type=jnp.float32)


def _topk_mean_cols(att_all, k_slots, n_iter=28):
    nrows, ncols = att_all.shape
    kf = jnp.float32(k_slots)
    ones_row = jnp.ones((1, nrows), jnp.float32)
    dn = (((1,), (0,)), ((), ()))

    def colsum(mat):
        return jax.lax.dot_general(ones_row, mat, dn,
                                   preferred_element_type=jnp.float32)

    def body(_, carry):
        lo, hi = carry
        mid = 0.5 * (lo + hi)
        cnt = colsum(jnp.where(att_all > mid, 1.0, 0.0))
        pred = cnt >= kf
        return jnp.where(pred, mid, lo), jnp.where(pred, hi, mid)

    lo0 = jnp.zeros((1, ncols), jnp.float32)
    hi0 = jnp.ones((1, ncols), jnp.float32)
    _, hi = jax.lax.fori_loop(0, n_iter, body, (lo0, hi0))
    gt = (att_all > hi).astype(jnp.float32)
    cnt_gt = colsum(gt)
    sum_gt = colsum(att_all * gt)
    return (sum_gt + (kf - cnt_gt) * hi) * (1.0 / kf)


def _topk_time_masks(vals, k_time, t, n_iter=34):
    n_sel = vals.shape[0]
    kf = jnp.float32(k_time)
    iota = jax.lax.broadcasted_iota(jnp.int32, (n_sel, t), 1)

    def body(_, carry):
        lo, hi = carry
        mid = 0.5 * (lo + hi)
        cnt = jnp.sum((vals > mid).astype(jnp.float32), axis=1, keepdims=True)
        pred = cnt >= kf
        return jnp.where(pred, mid, lo), jnp.where(pred, hi, mid)

    lo0 = jnp.zeros((n_sel, 1), jnp.float32)
    hi0 = jnp.ones((n_sel, 1), jnp.float32)
    _, hi = jax.lax.fori_loop(0, n_iter, body, (lo0, hi0))

    gt = vals > hi
    vstar = jnp.max(jnp.where(gt, -1.0, vals), axis=1, keepdims=True)
    mask = jnp.where(gt | (vals == vstar), 1.0, 0.0)
    count = jnp.sum(mask, axis=1, keepdims=True)

    def add_cond(carry):
        mask, count = carry
        return jnp.max(kf - count) > 0.0

    def add_body(carry):
        mask, count = carry
        rest = jnp.where(mask > 0.0, -1.0, vals)
        m = jnp.max(rest, axis=1, keepdims=True)
        cand = jnp.where((rest == m) & (mask == 0.0), iota, t)
        j = jnp.min(cand, axis=1, keepdims=True)
        grow = (iota == j) & (count < kf)
        mask = jnp.where(grow, 1.0, mask)
        return mask, jnp.sum(mask, axis=1, keepdims=True)

    def drop_cond(carry):
        mask, count = carry
        return jnp.max(count - kf) > 0.0

    def drop_body(carry):
        mask, count = carry
        cand = jnp.where((mask > 0.0) & (vals == vstar), iota, -1)
        j = jnp.max(cand, axis=1, keepdims=True)
        shrink = (iota == j) & (count > kf)
        mask = jnp.where(shrink, 0.0, mask)
        return mask, jnp.sum(mask, axis=1, keepdims=True)

    mask, count = jax.lax.while_loop(add_cond, add_body, (mask, count))
    mask, count = jax.lax.while_loop(drop_cond, drop_body, (mask, count))
    return mask


def _main_kernel(xn_ref, xa_ref, eps_ref, a_mem_ref, n_mem_ref,
                 g_a_ref, g_n_ref, h_n_ref, b_mu_ref, b_var_ref,
                 out_n_ref, out_a_ref, tatt_ref, acc_ref,
                 *, k_slots, k_time, t):
    i = pl.program_id(0)
    inv = jnp.float32(1.0 / (float(a_mem_ref.shape[1]) ** 0.5))

    xn = xn_ref[0]
    xa = xa_ref[0]
    eps = eps_ref[0]
    a_mem = a_mem_ref[...]
    n_mem = n_mem_ref[...]

    def att(mem, data):
        logits = jnp.dot(mem, data, preferred_element_type=jnp.float32) * inv
        return jax.nn.sigmoid(logits)

    att_aa = att(a_mem, xa)
    att_nn = att(n_mem, xn)
    att_na = att(a_mem, xn)
    att_an = att(n_mem, xa)

    b_mu = b_mu_ref[...]
    b_var = b_var_ref[...]
    g_a = g_a_ref[...]
    g_n = g_n_ref[...]

    def proj(g, a, b):
        return jnp.dot(g, a, preferred_element_type=jnp.float32) + b

    p_mu = proj(g_n, att_nn, b_mu)
    p_var = proj(h_n_ref[...], att_nn, b_var)
    a_proj = proj(g_a, att_aa, b_mu)
    na_proj = proj(g_a, att_na, b_mu)
    an_proj = proj(g_n, att_an, b_mu)

    expv = jnp.exp(p_var)
    std = jnp.sqrt(expv)
    n_aug_new = p_mu + eps * std
    out_n_ref[0] = n_aug_new + na_proj
    out_a_ref[0] = a_proj + an_proj

    kl_i = jnp.sum(1.0 + p_var - p_mu * p_mu - expv)

    att_all = jnp.concatenate([att_aa, att_nn, att_na, att_an], axis=1)
    tatt = _topk_mean_cols(att_all, k_slots, n_iter=28)
    tatt_ref[0] = tatt

    tsel = jnp.concatenate(
        [tatt[:, 0:t], tatt[:, t:2 * t], tatt[:, 3 * t:4 * t]], axis=0)
    masks = _topk_time_masks(tsel, k_time, t)
    m_a = masks[0:1, :]
    m_n = masks[1:2, :]
    m_p = masks[2:3, :]

    cmean = jnp.float32(1.0 / k_time)

    def selmean(mat, mask):
        return jnp.sum(mat * mask, axis=1, keepdims=True) * cmean

    negative_ax = selmean(xa, m_a)
    anchor_nx = selmean(xn, m_n)
    positive_nx = selmean(xa, m_p)
    anchor_new = selmean(n_aug_new, m_n)
    negative_new = selmean(a_proj, m_a)

    def l2n(v):
        return v / jnp.sqrt(jnp.sum(v * v))

    a_ = l2n(anchor_nx)
    p_ = l2n(positive_nx)
    n_ = l2n(negative_ax)
    dp = jnp.sqrt(jnp.sum((a_ - p_ + 1e-6) ** 2))
    dnn = jnp.sqrt(jnp.sum((a_ - n_ + 1e-6) ** 2))
    tml_i = jnp.maximum(dp - dnn + 1.0, 0.0)
    dist_i = jnp.maximum(
        100.0 - jnp.sqrt(jnp.sum(negative_new * negative_new))
        + jnp.sqrt(jnp.sum(anchor_new * anchor_new)), 0.0)

    rows = jax.lax.broadcasted_iota(jnp.int32, (8, 128), 0)
    cols = jax.lax.broadcasted_iota(jnp.int32, (8, 128), 1)
    contrib = (jnp.where((rows == 0) & (cols == 0), tml_i, 0.0)
               + jnp.where((rows == 0) & (cols == 1), dist_i, 0.0)
               + jnp.where((rows == 0) & (cols == 2), kl_i, 0.0))

    @pl.when(i == 0)
    def _():
        acc_ref[...] = jnp.zeros((8, 128), jnp.float32)

    acc_ref[...] += contrib


def kernel(x, A_memory, N_memory, W_mu, b_mu, W_var, b_var, epsilon):
    b, t, d = x.shape
    mid = b // 2
    K = A_memory.shape[0]
    k_slots = K // 16 + 1
    k_time = t // 16 + 1

    xt = jnp.transpose(x, (0, 2, 1))
    xn_t = xt[:mid]
    xa_t = xt[mid:]
    eps_t = jnp.transpose(epsilon, (0, 2, 1))

    g_a, g_n, h_n = pl.pallas_call(
        _prep_kernel,
        out_shape=[jax.ShapeDtypeStruct((d, K), jnp.float32)] * 3,
    )(A_memory, N_memory, W_mu, W_var)

    body = functools.partial(_main_kernel, k_slots=k_slots, k_time=k_time, t=t)
    out_n_t, out_a_t, tatt, acc = pl.pallas_call(
        body,
        grid=(mid,),
        in_specs=[
            pl.BlockSpec((1, d, t), lambda i: (i, 0, 0)),
            pl.BlockSpec((1, d, t), lambda i: (i, 0, 0)),
            pl.BlockSpec((1, d, t), lambda i: (i, 0, 0)),
            pl.BlockSpec((K, d), lambda i: (0, 0)),
            pl.BlockSpec((K, d), lambda i: (0, 0)),
            pl.BlockSpec((d, K), lambda i: (0, 0)),
            pl.BlockSpec((d, K), lambda i: (0, 0)),
            pl.BlockSpec((d, K), lambda i: (0, 0)),
            pl.BlockSpec((d, 1), lambda i: (0, 0)),
            pl.BlockSpec((d, 1), lambda i: (0, 0)),
        ],
        out_specs=[
            pl.BlockSpec((1, d, t), lambda i: (i, 0, 0)),
            pl.BlockSpec((1, d, t), lambda i: (i, 0, 0)),
            pl.BlockSpec((1, 1, 4 * t), lambda i: (i, 0, 0)),
            pl.BlockSpec((8, 128), lambda i: (0, 0)),
        ],
        out_shape=[
            jax.ShapeDtypeStruct((mid, d, t), jnp.float32),
            jax.ShapeDtypeStruct((mid, d, t), jnp.float32),
            jax.ShapeDtypeStruct((mid, 1, 4 * t), jnp.float32),
            jax.ShapeDtypeStruct((8, 128), jnp.float32),
        ],
    )(xn_t, xa_t, eps_t, A_memory, N_memory, g_a, g_n, h_n,
      b_mu.reshape(d, 1), b_var.reshape(d, 1))

    A_att = tatt[:, 0, 0:t]
    N_att = tatt[:, 0, t:2 * t]
    A_Natt = tatt[:, 0, 2 * t:3 * t]
    N_Aatt = tatt[:, 0, 3 * t:4 * t]

    right = jnp.concatenate(
        [jnp.transpose(out_n_t, (0, 2, 1)), jnp.transpose(out_a_t, (0, 2, 1))],
        axis=0)
    F_M = jnp.concatenate([x, right], axis=-1)

    tml = acc[0, 0] / mid
    distance = acc[0, 1] / mid
    kl_loss = -0.5 * acc[0, 2] / (mid * d)
    return (F_M, tml, kl_loss, distance, A_att, N_att, A_Natt, N_Aatt)


# speedup vs baseline: 1.0193x; 1.0193x over previous
import functools

import jax
import jax.numpy as jnp
from jax.experimental import pallas as pl


def _prep_kernel(a_mem_ref, n_mem_ref, w_mu_ref, w_var_ref,
                 g_a_ref, g_n_ref, h_n_ref):
    dn = (((1,), (1,)), ((), ()))
    g_a_ref[...] = jax.lax.dot_general(
        w_mu_ref[...], a_mem_ref[...], dn, preferred_element_type=jnp.float32)
    g_n_ref[...] = jax.lax.dot_general(
        w_mu_ref[...], n_mem_ref[...], dn, preferred_element_type=jnp.float32)
    h_n_ref[...] = jax.lax.dot_general(
        w_var_ref[...], n_mem_ref[...], dn, preferred_element_type=jnp.float32)


def _topk_mean_cols(att_all, k_slots, n_iter=28):
    nrows, ncols = att_all.shape
    kf = jnp.float32(k_slots)
    ones_row = jnp.ones((1, nrows), jnp.float32)
    dn = (((1,), (0,)), ((), ()))

    def colsum(mat):
        return jax.lax.dot_general(ones_row, mat, dn,
                                   preferred_element_type=jnp.float32)

    def body(_, carry):
        lo, hi = carry
        mid = 0.5 * (lo + hi)
        cnt = jnp.sum(jnp.where(att_all > mid, 1.0, 0.0), axis=0, keepdims=True)
        pred = cnt >= kf
        return jnp.where(pred, mid, lo), jnp.where(pred, hi, mid)

    lo0 = jnp.zeros((1, ncols), jnp.float32)
    hi0 = jnp.ones((1, ncols), jnp.float32)
    _, hi = jax.lax.fori_loop(0, n_iter, body, (lo0, hi0))
    gt = (att_all > hi).astype(jnp.float32)
    cnt_gt = colsum(gt)
    sum_gt = colsum(att_all * gt)
    return (sum_gt + (kf - cnt_gt) * hi) * (1.0 / kf)


def _topk_time_masks(vals, k_time, t, n_iter=34):
    n_sel = vals.shape[0]
    kf = jnp.float32(k_time)
    iota = jax.lax.broadcasted_iota(jnp.int32, (n_sel, t), 1)

    def body(_, carry):
        lo, hi = carry
        mid = 0.5 * (lo + hi)
        cnt = jnp.sum((vals > mid).astype(jnp.float32), axis=1, keepdims=True)
        pred = cnt >= kf
        return jnp.where(pred, mid, lo), jnp.where(pred, hi, mid)

    lo0 = jnp.zeros((n_sel, 1), jnp.float32)
    hi0 = jnp.ones((n_sel, 1), jnp.float32)
    _, hi = jax.lax.fori_loop(0, n_iter, body, (lo0, hi0))

    gt = vals > hi
    vstar = jnp.max(jnp.where(gt, -1.0, vals), axis=1, keepdims=True)
    mask = jnp.where(gt | (vals == vstar), 1.0, 0.0)
    count = jnp.sum(mask, axis=1, keepdims=True)

    def add_cond(carry):
        mask, count = carry
        return jnp.max(kf - count) > 0.0

    def add_body(carry):
        mask, count = carry
        rest = jnp.where(mask > 0.0, -1.0, vals)
        m = jnp.max(rest, axis=1, keepdims=True)
        cand = jnp.where((rest == m) & (mask == 0.0), iota, t)
        j = jnp.min(cand, axis=1, keepdims=True)
        grow = (iota == j) & (count < kf)
        mask = jnp.where(grow, 1.0, mask)
        return mask, jnp.sum(mask, axis=1, keepdims=True)

    def drop_cond(carry):
        mask, count = carry
        return jnp.max(count - kf) > 0.0

    def drop_body(carry):
        mask, count = carry
        cand = jnp.where((mask > 0.0) & (vals == vstar), iota, -1)
        j = jnp.max(cand, axis=1, keepdims=True)
        shrink = (iota == j) & (count > kf)
        mask = jnp.where(shrink, 0.0, mask)
        return mask, jnp.sum(mask, axis=1, keepdims=True)

    mask, count = jax.lax.while_loop(add_cond, add_body, (mask, count))
    mask, count = jax.lax.while_loop(drop_cond, drop_body, (mask, count))
    return mask


def _main_kernel(xn_ref, xa_ref, eps_ref, a_mem_ref, n_mem_ref,
                 g_a_ref, g_n_ref, h_n_ref, b_mu_ref, b_var_ref,
                 out_n_ref, out_a_ref, tatt_ref, acc_ref,
                 *, k_slots, k_time, t):
    i = pl.program_id(0)
    inv = jnp.float32(1.0 / (float(a_mem_ref.shape[1]) ** 0.5))

    xn = xn_ref[0]
    xa = xa_ref[0]
    eps = eps_ref[0]
    a_mem = a_mem_ref[...]
    n_mem = n_mem_ref[...]

    def att(mem, data):
        logits = jnp.dot(mem, data, preferred_element_type=jnp.float32) * inv
        return jax.nn.sigmoid(logits)

    att_aa = att(a_mem, xa)
    att_nn = att(n_mem, xn)
    att_na = att(a_mem, xn)
    att_an = att(n_mem, xa)

    b_mu = b_mu_ref[...]
    b_var = b_var_ref[...]
    g_a = g_a_ref[...]
    g_n = g_n_ref[...]

    def proj(g, a, b):
        return jnp.dot(g, a, preferred_element_type=jnp.float32) + b

    p_mu = proj(g_n, att_nn, b_mu)
    p_var = proj(h_n_ref[...], att_nn, b_var)
    a_proj = proj(g_a, att_aa, b_mu)
    na_proj = proj(g_a, att_na, b_mu)
    an_proj = proj(g_n, att_an, b_mu)

    expv = jnp.exp(p_var)
    std = jnp.sqrt(expv)
    n_aug_new = p_mu + eps * std
    out_n_ref[0] = n_aug_new + na_proj
    out_a_ref[0] = a_proj + an_proj

    kl_i = jnp.sum(1.0 + p_var - p_mu * p_mu - expv)

    att_all = jnp.concatenate([att_aa, att_nn, att_na, att_an], axis=1)
    tatt = _topk_mean_cols(att_all, k_slots, n_iter=28)
    tatt_ref[0] = tatt

    tsel = jnp.concatenate(
        [tatt[:, 0:t], tatt[:, t:2 * t], tatt[:, 3 * t:4 * t]], axis=0)
    masks = _topk_time_masks(tsel, k_time, t)
    m_a = masks[0:1, :]
    m_n = masks[1:2, :]
    m_p = masks[2:3, :]

    cmean = jnp.float32(1.0 / k_time)

    def selmean(mat, mask):
        return jnp.sum(mat * mask, axis=1, keepdims=True) * cmean

    negative_ax = selmean(xa, m_a)
    anchor_nx = selmean(xn, m_n)
    positive_nx = selmean(xa, m_p)
    anchor_new = selmean(n_aug_new, m_n)
    negative_new = selmean(a_proj, m_a)

    def l2n(v):
        return v / jnp.sqrt(jnp.sum(v * v))

    a_ = l2n(anchor_nx)
    p_ = l2n(positive_nx)
    n_ = l2n(negative_ax)
    dp = jnp.sqrt(jnp.sum((a_ - p_ + 1e-6) ** 2))
    dnn = jnp.sqrt(jnp.sum((a_ - n_ + 1e-6) ** 2))
    tml_i = jnp.maximum(dp - dnn + 1.0, 0.0)
    dist_i = jnp.maximum(
        100.0 - jnp.sqrt(jnp.sum(negative_new * negative_new))
        + jnp.sqrt(jnp.sum(anchor_new * anchor_new)), 0.0)

    rows = jax.lax.broadcasted_iota(jnp.int32, (8, 128), 0)
    cols = jax.lax.broadcasted_iota(jnp.int32, (8, 128), 1)
    contrib = (jnp.where((rows == 0) & (cols == 0), tml_i, 0.0)
               + jnp.where((rows == 0) & (cols == 1), dist_i, 0.0)
               + jnp.where((rows == 0) & (cols == 2), kl_i, 0.0))

    @pl.when(i == 0)
    def _():
        acc_ref[...] = jnp.zeros((8, 128), jnp.float32)

    acc_ref[...] += contrib


def kernel(x, A_memory, N_memory, W_mu, b_mu, W_var, b_var, epsilon):
    b, t, d = x.shape
    mid = b // 2
    K = A_memory.shape[0]
    k_slots = K // 16 + 1
    k_time = t // 16 + 1

    xt = jnp.transpose(x, (0, 2, 1))
    xn_t = xt[:mid]
    xa_t = xt[mid:]
    eps_t = jnp.transpose(epsilon, (0, 2, 1))

    g_a, g_n, h_n = pl.pallas_call(
        _prep_kernel,
        out_shape=[jax.ShapeDtypeStruct((d, K), jnp.float32)] * 3,
    )(A_memory, N_memory, W_mu, W_var)

    body = functools.partial(_main_kernel, k_slots=k_slots, k_time=k_time, t=t)
    out_n_t, out_a_t, tatt, acc = pl.pallas_call(
        body,
        grid=(mid,),
        in_specs=[
            pl.BlockSpec((1, d, t), lambda i: (i, 0, 0)),
            pl.BlockSpec((1, d, t), lambda i: (i, 0, 0)),
            pl.BlockSpec((1, d, t), lambda i: (i, 0, 0)),
            pl.BlockSpec((K, d), lambda i: (0, 0)),
            pl.BlockSpec((K, d), lambda i: (0, 0)),
            pl.BlockSpec((d, K), lambda i: (0, 0)),
            pl.BlockSpec((d, K), lambda i: (0, 0)),
            pl.BlockSpec((d, K), lambda i: (0, 0)),
            pl.BlockSpec((d, 1), lambda i: (0, 0)),
            pl.BlockSpec((d, 1), lambda i: (0, 0)),
        ],
        out_specs=[
            pl.BlockSpec((1, d, t), lambda i: (i, 0, 0)),
            pl.BlockSpec((1, d, t), lambda i: (i, 0, 0)),
            pl.BlockSpec((1, 1, 4 * t), lambda i: (i, 0, 0)),
            pl.BlockSpec((8, 128), lambda i: (0, 0)),
        ],
        out_shape=[
            jax.ShapeDtypeStruct((mid, d, t), jnp.float32),
            jax.ShapeDtypeStruct((mid, d, t), jnp.float32),
            jax.ShapeDtypeStruct((mid, 1, 4 * t), jnp.float32),
            jax.ShapeDtypeStruct((8, 128), jnp.float32),
        ],
    )(xn_t, xa_t, eps_t, A_memory, N_memory, g_a, g_n, h_n,
      b_mu.reshape(d, 1), b_var.reshape(d, 1))

    A_att = tatt[:, 0, 0:t]
    N_att = tatt[:, 0, t:2 * t]
    A_Natt = tatt[:, 0, 2 * t:3 * t]
    N_Aatt = tatt[:, 0, 3 * t:4 * t]

    right = jnp.concatenate(
        [jnp.transpose(out_n_t, (0, 2, 1)), jnp.transpose(out_a_t, (0, 2, 1))],
        axis=0)
    F_M = jnp.concatenate([x, right], axis=-1)

    tml = acc[0, 0] / mid
    distance = acc[0, 1] / mid
    kl_loss = -0.5 * acc[0, 2] / (mid * d)
    return (F_M, tml, kl_loss, distance, A_att, N_att, A_Natt, N_Aatt)


# speedup vs baseline: 1.0478x; 1.0280x over previous
import functools

import jax
import jax.numpy as jnp
from jax.experimental import pallas as pl


def _prep_kernel(a_mem_ref, n_mem_ref, w_mu_ref, w_var_ref,
                 g_a_ref, g_n_ref, h_n_ref):
    dn = (((1,), (1,)), ((), ()))
    g_a_ref[...] = jax.lax.dot_general(
        w_mu_ref[...], a_mem_ref[...], dn, preferred_element_type=jnp.float32)
    g_n_ref[...] = jax.lax.dot_general(
        w_mu_ref[...], n_mem_ref[...], dn, preferred_element_type=jnp.float32)
    h_n_ref[...] = jax.lax.dot_general(
        w_var_ref[...], n_mem_ref[...], dn, preferred_element_type=jnp.float32)


def _topk_mean_cols(att_all, k_slots, n_iter=28):
    nrows, ncols = att_all.shape
    kf = jnp.float32(k_slots)
    ones_row = jnp.ones((1, nrows), jnp.float32)
    dn = (((1,), (0,)), ((), ()))

    def colsum(mat):
        return jax.lax.dot_general(ones_row, mat, dn,
                                   preferred_element_type=jnp.float32)

    def body(_, carry):
        lo, hi = carry
        for _ in range(2):
            mid = 0.5 * (lo + hi)
            cnt = jnp.sum(jnp.where(att_all > mid, 1.0, 0.0), axis=0,
                          keepdims=True)
            pred = cnt >= kf
            lo = jnp.where(pred, mid, lo)
            hi = jnp.where(pred, hi, mid)
        return lo, hi

    lo0 = jnp.zeros((1, ncols), jnp.float32)
    hi0 = jnp.ones((1, ncols), jnp.float32)
    _, hi = jax.lax.fori_loop(0, n_iter // 2, body, (lo0, hi0))
    gt = (att_all > hi).astype(jnp.float32)
    cnt_gt = colsum(gt)
    sum_gt = colsum(att_all * gt)
    return (sum_gt + (kf - cnt_gt) * hi) * (1.0 / kf)


def _topk_time_masks(vals, k_time, t, n_iter=34):
    n_sel = vals.shape[0]
    kf = jnp.float32(k_time)
    iota = jax.lax.broadcasted_iota(jnp.int32, (n_sel, t), 1)

    def body(_, carry):
        lo, hi = carry
        mid = 0.5 * (lo + hi)
        cnt = jnp.sum((vals > mid).astype(jnp.float32), axis=1, keepdims=True)
        pred = cnt >= kf
        return jnp.where(pred, mid, lo), jnp.where(pred, hi, mid)

    lo0 = jnp.zeros((n_sel, 1), jnp.float32)
    hi0 = jnp.ones((n_sel, 1), jnp.float32)
    _, hi = jax.lax.fori_loop(0, n_iter, body, (lo0, hi0))

    gt = vals > hi
    vstar = jnp.max(jnp.where(gt, -1.0, vals), axis=1, keepdims=True)
    mask = jnp.where(gt | (vals == vstar), 1.0, 0.0)
    count = jnp.sum(mask, axis=1, keepdims=True)

    def add_cond(carry):
        mask, count = carry
        return jnp.max(kf - count) > 0.0

    def add_body(carry):
        mask, count = carry
        rest = jnp.where(mask > 0.0, -1.0, vals)
        m = jnp.max(rest, axis=1, keepdims=True)
        cand = jnp.where((rest == m) & (mask == 0.0), iota, t)
        j = jnp.min(cand, axis=1, keepdims=True)
        grow = (iota == j) & (count < kf)
        mask = jnp.where(grow, 1.0, mask)
        return mask, jnp.sum(mask, axis=1, keepdims=True)

    def drop_cond(carry):
        mask, count = carry
        return jnp.max(count - kf) > 0.0

    def drop_body(carry):
        mask, count = carry
        cand = jnp.where((mask > 0.0) & (vals == vstar), iota, -1)
        j = jnp.max(cand, axis=1, keepdims=True)
        shrink = (iota == j) & (count > kf)
        mask = jnp.where(shrink, 0.0, mask)
        return mask, jnp.sum(mask, axis=1, keepdims=True)

    mask, count = jax.lax.while_loop(add_cond, add_body, (mask, count))
    mask, count = jax.lax.while_loop(drop_cond, drop_body, (mask, count))
    return mask


def _main_kernel(xn_ref, xa_ref, eps_ref, a_mem_ref, n_mem_ref,
                 g_a_ref, g_n_ref, h_n_ref, b_mu_ref, b_var_ref,
                 out_n_ref, out_a_ref, tatt_ref, acc_ref,
                 *, k_slots, k_time, t):
    i = pl.program_id(0)
    inv = jnp.float32(1.0 / (float(a_mem_ref.shape[1]) ** 0.5))

    xn = xn_ref[0]
    xa = xa_ref[0]
    eps = eps_ref[0]
    a_mem = a_mem_ref[...]
    n_mem = n_mem_ref[...]

    def att(mem, data):
        logits = jnp.dot(mem, data, preferred_element_type=jnp.float32) * inv
        return jax.nn.sigmoid(logits)

    att_aa = att(a_mem, xa)
    att_nn = att(n_mem, xn)
    att_na = att(a_mem, xn)
    att_an = att(n_mem, xa)

    b_mu = b_mu_ref[...]
    b_var = b_var_ref[...]
    g_a = g_a_ref[...]
    g_n = g_n_ref[...]

    def proj(g, a, b):
        return jnp.dot(g, a, preferred_element_type=jnp.float32) + b

    p_mu = proj(g_n, att_nn, b_mu)
    p_var = proj(h_n_ref[...], att_nn, b_var)
    a_proj = proj(g_a, att_aa, b_mu)
    na_proj = proj(g_a, att_na, b_mu)
    an_proj = proj(g_n, att_an, b_mu)

    expv = jnp.exp(p_var)
    std = jnp.sqrt(expv)
    n_aug_new = p_mu + eps * std
    out_n_ref[0] = n_aug_new + na_proj
    out_a_ref[0] = a_proj + an_proj

    kl_i = jnp.sum(1.0 + p_var - p_mu * p_mu - expv)

    att_all = jnp.concatenate([att_aa, att_nn, att_na, att_an], axis=1)
    tatt = _topk_mean_cols(att_all, k_slots, n_iter=28)
    tatt_ref[0] = tatt

    tsel = jnp.concatenate(
        [tatt[:, 0:t], tatt[:, t:2 * t], tatt[:, 3 * t:4 * t]], axis=0)
    masks = _topk_time_masks(tsel, k_time, t)

    cmean = jnp.float32(1.0 / k_time)
    dn_t = (((1,), (1,)), ((), ()))

    def selmeans(mat):
        return jax.lax.dot_general(masks, mat, dn_t,
                                   preferred_element_type=jnp.float32) * cmean

    v_xa = selmeans(xa)
    v_xn = selmeans(xn)
    v_naug = selmeans(n_aug_new)
    v_aproj = selmeans(a_proj)

    negative_ax = v_xa[0:1, :]
    anchor_nx = v_xn[1:2, :]
    positive_nx = v_xa[2:3, :]
    anchor_new = v_naug[1:2, :]
    negative_new = v_aproj[0:1, :]

    def l2n(v):
        return v / jnp.sqrt(jnp.sum(v * v))

    a_ = l2n(anchor_nx)
    p_ = l2n(positive_nx)
    n_ = l2n(negative_ax)
    dp = jnp.sqrt(jnp.sum((a_ - p_ + 1e-6) ** 2))
    dnn = jnp.sqrt(jnp.sum((a_ - n_ + 1e-6) ** 2))
    tml_i = jnp.maximum(dp - dnn + 1.0, 0.0)
    dist_i = jnp.maximum(
        100.0 - jnp.sqrt(jnp.sum(negative_new * negative_new))
        + jnp.sqrt(jnp.sum(anchor_new * anchor_new)), 0.0)

    rows = jax.lax.broadcasted_iota(jnp.int32, (8, 128), 0)
    cols = jax.lax.broadcasted_iota(jnp.int32, (8, 128), 1)
    contrib = (jnp.where((rows == 0) & (cols == 0), tml_i, 0.0)
               + jnp.where((rows == 0) & (cols == 1), dist_i, 0.0)
               + jnp.where((rows == 0) & (cols == 2), kl_i, 0.0))

    @pl.when(i == 0)
    def _():
        acc_ref[...] = jnp.zeros((8, 128), jnp.float32)

    acc_ref[...] += contrib


def kernel(x, A_memory, N_memory, W_mu, b_mu, W_var, b_var, epsilon):
    b, t, d = x.shape
    mid = b // 2
    K = A_memory.shape[0]
    k_slots = K // 16 + 1
    k_time = t // 16 + 1

    xt = jnp.transpose(x, (0, 2, 1))
    xn_t = xt[:mid]
    xa_t = xt[mid:]
    eps_t = jnp.transpose(epsilon, (0, 2, 1))

    g_a, g_n, h_n = pl.pallas_call(
        _prep_kernel,
        out_shape=[jax.ShapeDtypeStruct((d, K), jnp.float32)] * 3,
    )(A_memory, N_memory, W_mu, W_var)

    body = functools.partial(_main_kernel, k_slots=k_slots, k_time=k_time, t=t)
    out_n_t, out_a_t, tatt, acc = pl.pallas_call(
        body,
        grid=(mid,),
        in_specs=[
            pl.BlockSpec((1, d, t), lambda i: (i, 0, 0)),
            pl.BlockSpec((1, d, t), lambda i: (i, 0, 0)),
            pl.BlockSpec((1, d, t), lambda i: (i, 0, 0)),
            pl.BlockSpec((K, d), lambda i: (0, 0)),
            pl.BlockSpec((K, d), lambda i: (0, 0)),
            pl.BlockSpec((d, K), lambda i: (0, 0)),
            pl.BlockSpec((d, K), lambda i: (0, 0)),
            pl.BlockSpec((d, K), lambda i: (0, 0)),
            pl.BlockSpec((d, 1), lambda i: (0, 0)),
            pl.BlockSpec((d, 1), lambda i: (0, 0)),
        ],
        out_specs=[
            pl.BlockSpec((1, d, t), lambda i: (i, 0, 0)),
            pl.BlockSpec((1, d, t), lambda i: (i, 0, 0)),
            pl.BlockSpec((1, 1, 4 * t), lambda i: (i, 0, 0)),
            pl.BlockSpec((8, 128), lambda i: (0, 0)),
        ],
        out_shape=[
            jax.ShapeDtypeStruct((mid, d, t), jnp.float32),
            jax.ShapeDtypeStruct((mid, d, t), jnp.float32),
            jax.ShapeDtypeStruct((mid, 1, 4 * t), jnp.float32),
            jax.ShapeDtypeStruct((8, 128), jnp.float32),
        ],
    )(xn_t, xa_t, eps_t, A_memory, N_memory, g_a, g_n, h_n,
      b_mu.reshape(d, 1), b_var.reshape(d, 1))

    A_att = tatt[:, 0, 0:t]
    N_att = tatt[:, 0, t:2 * t]
    A_Natt = tatt[:, 0, 2 * t:3 * t]
    N_Aatt = tatt[:, 0, 3 * t:4 * t]

    right = jnp.concatenate(
        [jnp.transpose(out_n_t, (0, 2, 1)), jnp.transpose(out_a_t, (0, 2, 1))],
        axis=0)
    F_M = jnp.concatenate([x, right], axis=-1)

    tml = acc[0, 0] / mid
    distance = acc[0, 1] / mid
    kl_loss = -0.5 * acc[0, 2] / (mid * d)
    return (F_M, tml, kl_loss, distance, A_att, N_att, A_Natt, N_Aatt)


# speedup vs baseline: 1.3017x; 1.2423x over previous
import functools

import jax
import jax.numpy as jnp
from jax.experimental import pallas as pl

_DN_FEAT = (((1,), (1,)), ((), ()))
_DN_SLOT = (((0,), (0,)), ((), ()))
_DN_TIME = (((1,), (0,)), ((), ()))


def _prep_kernel(a_mem_ref, n_mem_ref, w_mu_ref, w_var_ref,
                 m_amu_ref, m_nmu_ref, m_nvar_ref):
    m_amu_ref[...] = jax.lax.dot_general(
        a_mem_ref[...], w_mu_ref[...], _DN_FEAT,
        preferred_element_type=jnp.float32)
    m_nmu_ref[...] = jax.lax.dot_general(
        n_mem_ref[...], w_mu_ref[...], _DN_FEAT,
        preferred_element_type=jnp.float32)
    m_nvar_ref[...] = jax.lax.dot_general(
        n_mem_ref[...], w_var_ref[...], _DN_FEAT,
        preferred_element_type=jnp.float32)


def _topk_mean_cols(att_all, k_slots, n_iter=28):
    nrows, ncols = att_all.shape
    kf = jnp.float32(k_slots)
    ones_row = jnp.ones((1, nrows), jnp.float32)

    def colsum(mat):
        return jax.lax.dot_general(ones_row, mat, _DN_TIME,
                                   preferred_element_type=jnp.float32)

    def body(_, carry):
        lo, hi = carry
        for _ in range(2):
            mid = 0.5 * (lo + hi)
            cnt = jnp.sum(jnp.where(att_all > mid, 1.0, 0.0), axis=0,
                          keepdims=True)
            pred = cnt >= kf
            lo = jnp.where(pred, mid, lo)
            hi = jnp.where(pred, hi, mid)
        return lo, hi

    lo0 = jnp.zeros((1, ncols), jnp.float32)
    hi0 = jnp.ones((1, ncols), jnp.float32)
    _, hi = jax.lax.fori_loop(0, n_iter // 2, body, (lo0, hi0))
    gt = (att_all > hi).astype(jnp.float32)
    cnt_gt = colsum(gt)
    sum_gt = colsum(att_all * gt)
    return (sum_gt + (kf - cnt_gt) * hi) * (1.0 / kf)


def _topk_time_masks(vals, k_time, t, n_iter=34):
    n_sel = vals.shape[0]
    kf = jnp.float32(k_time)
    iota = jax.lax.broadcasted_iota(jnp.int32, (n_sel, t), 1)

    def body(_, carry):
        lo, hi = carry
        mid = 0.5 * (lo + hi)
        cnt = jnp.sum((vals > mid).astype(jnp.float32), axis=1, keepdims=True)
        pred = cnt >= kf
        return jnp.where(pred, mid, lo), jnp.where(pred, hi, mid)

    lo0 = jnp.zeros((n_sel, 1), jnp.float32)
    hi0 = jnp.ones((n_sel, 1), jnp.float32)
    _, hi = jax.lax.fori_loop(0, n_iter, body, (lo0, hi0))

    gt = vals > hi
    vstar = jnp.max(jnp.where(gt, -1.0, vals), axis=1, keepdims=True)
    mask = jnp.where(gt | (vals == vstar), 1.0, 0.0)
    count = jnp.sum(mask, axis=1, keepdims=True)

    def add_cond(carry):
        mask, count = carry
        return jnp.max(kf - count) > 0.0

    def add_body(carry):
        mask, count = carry
        rest = jnp.where(mask > 0.0, -1.0, vals)
        m = jnp.max(rest, axis=1, keepdims=True)
        cand = jnp.where((rest == m) & (mask == 0.0), iota, t)
        j = jnp.min(cand, axis=1, keepdims=True)
        grow = (iota == j) & (count < kf)
        mask = jnp.where(grow, 1.0, mask)
        return mask, jnp.sum(mask, axis=1, keepdims=True)

    def drop_cond(carry):
        mask, count = carry
        return jnp.max(count - kf) > 0.0

    def drop_body(carry):
        mask, count = carry
        cand = jnp.where((mask > 0.0) & (vals == vstar), iota, -1)
        j = jnp.max(cand, axis=1, keepdims=True)
        shrink = (iota == j) & (count > kf)
        mask = jnp.where(shrink, 0.0, mask)
        return mask, jnp.sum(mask, axis=1, keepdims=True)

    mask, count = jax.lax.while_loop(add_cond, add_body, (mask, count))
    mask, count = jax.lax.while_loop(drop_cond, drop_body, (mask, count))
    return mask


def _main_kernel(xn_ref, xa_ref, xnt_ref, xat_ref, eps_ref,
                 a_mem_ref, n_mem_ref,
                 m_amu_ref, m_nmu_ref, m_nvar_ref, b_mu_ref, b_var_ref,
                 f_ref, tatt_ref, acc_ref,
                 *, k_slots, k_time, t, d):
    i = pl.program_id(0)
    inv = jnp.float32(1.0 / (float(d) ** 0.5))

    xn = xn_ref[0]
    xa = xa_ref[0]
    xnt = xnt_ref[0]
    xat = xat_ref[0]
    eps = eps_ref[0]
    a_mem = a_mem_ref[...]
    n_mem = n_mem_ref[...]

    def att(mem, data_t):
        logits = jnp.dot(mem, data_t, preferred_element_type=jnp.float32) * inv
        return jax.nn.sigmoid(logits)

    att_aa = att(a_mem, xat)
    att_nn = att(n_mem, xnt)
    att_na = att(a_mem, xnt)
    att_an = att(n_mem, xat)

    b_mu = b_mu_ref[...]
    b_var = b_var_ref[...]

    def proj(a, m, b):
        return jax.lax.dot_general(
            a, m, _DN_SLOT, preferred_element_type=jnp.float32) + b

    p_mu = proj(att_nn, m_nmu_ref[...], b_mu)
    p_var = proj(att_nn, m_nvar_ref[...], b_var)
    a_proj = proj(att_aa, m_amu_ref[...], b_mu)
    na_proj = proj(att_na, m_amu_ref[...], b_mu)
    an_proj = proj(att_an, m_nmu_ref[...], b_mu)

    expv = jnp.exp(p_var)
    std = jnp.sqrt(expv)
    n_aug_new = p_mu + eps * std
    f_ref[0, 0, :, 0:d] = xn
    f_ref[0, 0, :, d:2 * d] = n_aug_new + na_proj
    f_ref[1, 0, :, 0:d] = xa
    f_ref[1, 0, :, d:2 * d] = a_proj + an_proj

    kl_i = jnp.sum(1.0 + p_var - p_mu * p_mu - expv)

    att_all = jnp.concatenate([att_aa, att_nn, att_na, att_an], axis=1)
    tatt = _topk_mean_cols(att_all, k_slots)
    tatt_ref[0] = tatt

    tsel = jnp.concatenate(
        [tatt[:, 0:t], tatt[:, t:2 * t], tatt[:, 3 * t:4 * t]], axis=0)
    masks = _topk_time_masks(tsel, k_time, t)

    cmean = jnp.float32(1.0 / k_time)

    def selmeans(mat):
        return jax.lax.dot_general(masks, mat, _DN_TIME,
                                   preferred_element_type=jnp.float32) * cmean

    v_xa = selmeans(xa)
    v_xn = selmeans(xn)
    v_naug = selmeans(n_aug_new)
    v_aproj = selmeans(a_proj)

    negative_ax = v_xa[0:1, :]
    anchor_nx = v_xn[1:2, :]
    positive_nx = v_xa[2:3, :]
    anchor_new = v_naug[1:2, :]
    negative_new = v_aproj[0:1, :]

    def l2n(v):
        return v / jnp.sqrt(jnp.sum(v * v))

    a_ = l2n(anchor_nx)
    p_ = l2n(positive_nx)
    n_ = l2n(negative_ax)
    dp = jnp.sqrt(jnp.sum((a_ - p_ + 1e-6) ** 2))
    dnn = jnp.sqrt(jnp.sum((a_ - n_ + 1e-6) ** 2))
    tml_i = jnp.maximum(dp - dnn + 1.0, 0.0)
    dist_i = jnp.maximum(
        100.0 - jnp.sqrt(jnp.sum(negative_new * negative_new))
        + jnp.sqrt(jnp.sum(anchor_new * anchor_new)), 0.0)

    rows = jax.lax.broadcasted_iota(jnp.int32, (8, 128), 0)
    cols = jax.lax.broadcasted_iota(jnp.int32, (8, 128), 1)
    contrib = (jnp.where((rows == 0) & (cols == 0), tml_i, 0.0)
               + jnp.where((rows == 0) & (cols == 1), dist_i, 0.0)
               + jnp.where((rows == 0) & (cols == 2), kl_i, 0.0))

    @pl.when(i == 0)
    def _():
        acc_ref[...] = jnp.zeros((8, 128), jnp.float32)

    acc_ref[...] += contrib


def kernel(x, A_memory, N_memory, W_mu, b_mu, W_var, b_var, epsilon):
    b, t, d = x.shape
    mid = b // 2
    K = A_memory.shape[0]
    k_slots = K // 16 + 1
    k_time = t // 16 + 1

    x_t = jnp.transpose(x, (0, 2, 1))

    m_amu, m_nmu, m_nvar = pl.pallas_call(
        _prep_kernel,
        out_shape=[jax.ShapeDtypeStruct((K, d), jnp.float32)] * 3,
    )(A_memory, N_memory, W_mu, W_var)

    body = functools.partial(_main_kernel, k_slots=k_slots, k_time=k_time,
                             t=t, d=d)
    f_halves, tatt, acc = pl.pallas_call(
        body,
        grid=(mid,),
        in_specs=[
            pl.BlockSpec((1, t, d), lambda i: (i, 0, 0)),
            pl.BlockSpec((1, t, d), lambda i: (i + mid, 0, 0)),
            pl.BlockSpec((1, d, t), lambda i: (i, 0, 0)),
            pl.BlockSpec((1, d, t), lambda i: (i + mid, 0, 0)),
            pl.BlockSpec((1, t, d), lambda i: (i, 0, 0)),
            pl.BlockSpec((K, d), lambda i: (0, 0)),
            pl.BlockSpec((K, d), lambda i: (0, 0)),
            pl.BlockSpec((K, d), lambda i: (0, 0)),
            pl.BlockSpec((K, d), lambda i: (0, 0)),
            pl.BlockSpec((K, d), lambda i: (0, 0)),
            pl.BlockSpec((1, d), lambda i: (0, 0)),
            pl.BlockSpec((1, d), lambda i: (0, 0)),
        ],
        out_specs=[
            pl.BlockSpec((2, 1, t, 2 * d), lambda i: (0, i, 0, 0)),
            pl.BlockSpec((1, 1, 4 * t), lambda i: (i, 0, 0)),
            pl.BlockSpec((8, 128), lambda i: (0, 0)),
        ],
        out_shape=[
            jax.ShapeDtypeStruct((2, mid, t, 2 * d), jnp.float32),
            jax.ShapeDtypeStruct((mid, 1, 4 * t), jnp.float32),
            jax.ShapeDtypeStruct((8, 128), jnp.float32),
        ],
    )(x, x, x_t, x_t, epsilon, A_memory, N_memory, m_amu, m_nmu, m_nvar,
      b_mu.reshape(1, d), b_var.reshape(1, d))

    A_att = tatt[:, 0, 0:t]
    N_att = tatt[:, 0, t:2 * t]
    A_Natt = tatt[:, 0, 2 * t:3 * t]
    N_Aatt = tatt[:, 0, 3 * t:4 * t]

    F_M = f_halves.reshape(b, t, 2 * d)

    tml = acc[0, 0] / mid
    distance = acc[0, 1] / mid
    kl_loss = -0.5 * acc[0, 2] / (mid * d)
    return (F_M, tml, kl_loss, distance, A_att, N_att, A_Natt, N_Aatt)


# speedup vs baseline: 1.3179x; 1.0125x over previous
import functools

import jax
import jax.numpy as jnp
from jax.experimental import pallas as pl

_DN_FEAT = (((1,), (1,)), ((), ()))
_DN_SLOT = (((0,), (0,)), ((), ()))
_DN_TIME = (((1,), (0,)), ((), ()))


def _prep_kernel(a_mem_ref, n_mem_ref, w_mu_ref, w_var_ref,
                 m_amu_ref, m_nmu_ref, m_nvar_ref):
    m_amu_ref[...] = jax.lax.dot_general(
        a_mem_ref[...], w_mu_ref[...], _DN_FEAT,
        preferred_element_type=jnp.float32).astype(jnp.bfloat16)
    m_nmu_ref[...] = jax.lax.dot_general(
        n_mem_ref[...], w_mu_ref[...], _DN_FEAT,
        preferred_element_type=jnp.float32).astype(jnp.bfloat16)
    m_nvar_ref[...] = jax.lax.dot_general(
        n_mem_ref[...], w_var_ref[...], _DN_FEAT,
        preferred_element_type=jnp.float32).astype(jnp.bfloat16)


def _topk_mean_cols(att_all, k_slots, n_iter=28):
    nrows, ncols = att_all.shape
    kf = jnp.float32(k_slots)

    def body(_, carry):
        lo, hi = carry
        for _ in range(2):
            mid = 0.5 * (lo + hi)
            cnt = jnp.sum(jnp.where(att_all > mid, 1.0, 0.0), axis=0,
                          keepdims=True)
            pred = cnt >= kf
            lo = jnp.where(pred, mid, lo)
            hi = jnp.where(pred, hi, mid)
        return lo, hi

    lo0 = jnp.zeros((1, ncols), jnp.float32)
    hi0 = jnp.ones((1, ncols), jnp.float32)
    _, hi = jax.lax.fori_loop(0, n_iter // 2, body, (lo0, hi0))
    gt = (att_all > hi).astype(jnp.float32)
    cnt_gt = jnp.sum(gt, axis=0, keepdims=True)
    sum_gt = jnp.sum(att_all * gt, axis=0, keepdims=True)
    return (sum_gt + (kf - cnt_gt) * hi) / kf


def _topk_time_masks(vals, k_time, t, n_iter=34):
    n_sel = vals.shape[0]
    kf = jnp.float32(k_time)
    iota = jax.lax.broadcasted_iota(jnp.int32, (n_sel, t), 1)

    def body(_, carry):
        lo, hi = carry
        mid = 0.5 * (lo + hi)
        cnt = jnp.sum((vals > mid).astype(jnp.float32), axis=1, keepdims=True)
        pred = cnt >= kf
        return jnp.where(pred, mid, lo), jnp.where(pred, hi, mid)

    lo0 = jnp.zeros((n_sel, 1), jnp.float32)
    hi0 = jnp.ones((n_sel, 1), jnp.float32)
    _, hi = jax.lax.fori_loop(0, n_iter, body, (lo0, hi0))

    gt = vals > hi
    vstar = jnp.max(jnp.where(gt, -1.0, vals), axis=1, keepdims=True)
    mask = jnp.where(gt | (vals == vstar), 1.0, 0.0)
    count = jnp.sum(mask, axis=1, keepdims=True)

    def add_cond(carry):
        mask, count = carry
        return jnp.max(kf - count) > 0.0

    def add_body(carry):
        mask, count = carry
        rest = jnp.where(mask > 0.0, -1.0, vals)
        m = jnp.max(rest, axis=1, keepdims=True)
        cand = jnp.where((rest == m) & (mask == 0.0), iota, t)
        j = jnp.min(cand, axis=1, keepdims=True)
        grow = (iota == j) & (count < kf)
        mask = jnp.where(grow, 1.0, mask)
        return mask, jnp.sum(mask, axis=1, keepdims=True)

    def drop_cond(carry):
        mask, count = carry
        return jnp.max(count - kf) > 0.0

    def drop_body(carry):
        mask, count = carry
        cand = jnp.where((mask > 0.0) & (vals == vstar), iota, -1)
        j = jnp.max(cand, axis=1, keepdims=True)
        shrink = (iota == j) & (count > kf)
        mask = jnp.where(shrink, 0.0, mask)
        return mask, jnp.sum(mask, axis=1, keepdims=True)

    mask, count = jax.lax.while_loop(add_cond, add_body, (mask, count))
    mask, count = jax.lax.while_loop(drop_cond, drop_body, (mask, count))
    return mask


def _main_kernel(xn_ref, xa_ref, xnb_ref, xab_ref, eps_ref,
                 a_memb_ref, n_memb_ref,
                 m_amu_ref, m_nmu_ref, m_nvar_ref, b_mu_ref, b_var_ref,
                 f_ref, tatt_ref, acc_ref,
                 *, k_slots, k_time, t, d):
    i = pl.program_id(0)
    sqrt_d = jnp.float32(float(d) ** 0.5)

    xn = xn_ref[0]
    xa = xa_ref[0]
    xnb = xnb_ref[0]
    xab = xab_ref[0]
    eps = eps_ref[0]
    a_memb = a_memb_ref[...]
    n_memb = n_memb_ref[...]

    def att(data_b, mem_b):
        logits = jax.lax.dot_general(
            data_b, mem_b, _DN_FEAT, preferred_element_type=jnp.float32)
        return jax.nn.sigmoid(logits / sqrt_d)

    att_aa = att(xab, a_memb)
    att_nn = att(xnb, n_memb)
    att_na = att(xnb, a_memb)
    att_an = att(xab, n_memb)

    b_mu = b_mu_ref[...]
    b_var = b_var_ref[...]

    def proj(a, m_ref, b):
        return jax.lax.dot_general(
            a.astype(jnp.bfloat16), m_ref[...], _DN_TIME,
            preferred_element_type=jnp.float32) + b

    p_mu = proj(att_nn, m_nmu_ref, b_mu)
    p_var = proj(att_nn, m_nvar_ref, b_var)
    a_proj = proj(att_aa, m_amu_ref, b_mu)
    na_proj = proj(att_na, m_amu_ref, b_mu)
    an_proj = proj(att_an, m_nmu_ref, b_mu)

    expv = jnp.exp(p_var)
    std = jnp.sqrt(expv)
    n_aug_new = p_mu + eps * std
    f_ref[0, 0, :, 0:d] = xn
    f_ref[0, 0, :, d:2 * d] = n_aug_new + na_proj
    f_ref[1, 0, :, 0:d] = xa
    f_ref[1, 0, :, d:2 * d] = a_proj + an_proj

    kl_i = jnp.sum(1.0 + p_var - p_mu * p_mu - expv)

    att_all = jnp.concatenate(
        [att_aa.T, att_nn.T, att_na.T, att_an.T], axis=1)
    tatt = _topk_mean_cols(att_all, k_slots)
    tatt_ref[0] = tatt

    tsel = jnp.concatenate(
        [tatt[:, 0:t], tatt[:, t:2 * t], tatt[:, 3 * t:4 * t]], axis=0)
    masks = _topk_time_masks(tsel, k_time, t)

    cmean = jnp.float32(1.0 / k_time)

    def selmeans(mat):
        return jax.lax.dot_general(
            masks, mat, _DN_TIME, precision=jax.lax.Precision.HIGHEST,
            preferred_element_type=jnp.float32) / jnp.float32(k_time)

    v_xa = selmeans(xa)
    v_xn = selmeans(xn)
    v_naug = selmeans(n_aug_new)
    v_aproj = selmeans(a_proj)

    negative_ax = v_xa[0:1, :]
    anchor_nx = v_xn[1:2, :]
    positive_nx = v_xa[2:3, :]
    anchor_new = v_naug[1:2, :]
    negative_new = v_aproj[0:1, :]

    def l2n(v):
        return v / jnp.sqrt(jnp.sum(v * v))

    a_ = l2n(anchor_nx)
    p_ = l2n(positive_nx)
    n_ = l2n(negative_ax)
    dp = jnp.sqrt(jnp.sum((a_ - p_ + 1e-6) ** 2))
    dnn = jnp.sqrt(jnp.sum((a_ - n_ + 1e-6) ** 2))
    tml_i = jnp.maximum(dp - dnn + 1.0, 0.0)
    dist_i = jnp.maximum(
        100.0 - jnp.sqrt(jnp.sum(negative_new * negative_new))
        + jnp.sqrt(jnp.sum(anchor_new * anchor_new)), 0.0)

    rows = jax.lax.broadcasted_iota(jnp.int32, (8, 128), 0)
    cols = jax.lax.broadcasted_iota(jnp.int32, (8, 128), 1)
    contrib = (jnp.where((rows == 0) & (cols == 0), tml_i, 0.0)
               + jnp.where((rows == 0) & (cols == 1), dist_i, 0.0)
               + jnp.where((rows == 0) & (cols == 2), kl_i, 0.0))

    @pl.when(i == 0)
    def _():
        acc_ref[...] = jnp.zeros((8, 128), jnp.float32)

    acc_ref[...] += contrib


def kernel(x, A_memory, N_memory, W_mu, b_mu, W_var, b_var, epsilon):
    b, t, d = x.shape
    mid = b // 2
    K = A_memory.shape[0]
    k_slots = K // 16 + 1
    k_time = t // 16 + 1

    x_b = x.astype(jnp.bfloat16)
    a_mem_b = A_memory.astype(jnp.bfloat16)
    n_mem_b = N_memory.astype(jnp.bfloat16)

    m_amu, m_nmu, m_nvar = pl.pallas_call(
        _prep_kernel,
        out_shape=[jax.ShapeDtypeStruct((K, d), jnp.bfloat16)] * 3,
    )(A_memory, N_memory, W_mu, W_var)

    body = functools.partial(_main_kernel, k_slots=k_slots, k_time=k_time,
                             t=t, d=d)
    f_halves, tatt, acc = pl.pallas_call(
        body,
        grid=(mid,),
        in_specs=[
            pl.BlockSpec((1, t, d), lambda i: (i, 0, 0)),
            pl.BlockSpec((1, t, d), lambda i: (i + mid, 0, 0)),
            pl.BlockSpec((1, t, d), lambda i: (i, 0, 0)),
            pl.BlockSpec((1, t, d), lambda i: (i + mid, 0, 0)),
            pl.BlockSpec((1, t, d), lambda i: (i, 0, 0)),
            pl.BlockSpec((K, d), lambda i: (0, 0)),
            pl.BlockSpec((K, d), lambda i: (0, 0)),
            pl.BlockSpec((K, d), lambda i: (0, 0)),
            pl.BlockSpec((K, d), lambda i: (0, 0)),
            pl.BlockSpec((K, d), lambda i: (0, 0)),
            pl.BlockSpec((1, d), lambda i: (0, 0)),
            pl.BlockSpec((1, d), lambda i: (0, 0)),
        ],
        out_specs=[
            pl.BlockSpec((2, 1, t, 2 * d), lambda i: (0, i, 0, 0)),
            pl.BlockSpec((1, 1, 4 * t), lambda i: (i, 0, 0)),
            pl.BlockSpec((8, 128), lambda i: (0, 0)),
        ],
        out_shape=[
            jax.ShapeDtypeStruct((2, mid, t, 2 * d), jnp.float32),
            jax.ShapeDtypeStruct((mid, 1, 4 * t), jnp.float32),
            jax.ShapeDtypeStruct((8, 128), jnp.float32),
        ],
    )(x, x, x_b, x_b, epsilon, a_mem_b, n_mem_b, m_amu, m_nmu, m_nvar,
      b_mu.reshape(1, d), b_var.reshape(1, d))

    A_att = tatt[:, 0, 0:t]
    N_att = tatt[:, 0, t:2 * t]
    A_Natt = tatt[:, 0, 2 * t:3 * t]
    N_Aatt = tatt[:, 0, 3 * t:4 * t]

    F_M = f_halves.reshape(b, t, 2 * d)

    tml = acc[0, 0] / mid
    distance = acc[0, 1] / mid
    kl_loss = -0.5 * acc[0, 2] / (mid * d)
    return (F_M, tml, kl_loss, distance, A_att, N_att, A_Natt, N_Aatt)
